# baseline (speedup 1.0000x reference)
import functools

import jax
import jax.numpy as jnp
from jax import lax
from jax.experimental import pallas as pl
from jax.experimental.pallas import tpu as pltpu
from jax.experimental.pallas import tpu_sc as plsc

EMB = 32
NROWS = 1000000
BATCH = 16384
NW = 32
B_PER_W = BATCH // NW
NCHUNK = 4
CHUNK = B_PER_W // NCHUNK
LANES = 16

_BR = 2048
_TC_GRID = (NROWS + _BR - 1) // _BR


def _detile_body(x_ref, o_ref):
    t = jnp.swapaxes(x_ref[...], 0, 1)
    t3 = t.reshape(_BR // 4, 4, EMB)
    o_ref[...] = jnp.concatenate(
        [t3[:, 0], t3[:, 1], t3[:, 2], t3[:, 3]], axis=1)


def _pack_table(table):
    return pl.pallas_call(
        _detile_body,
        grid=(_TC_GRID,),
        in_specs=[pl.BlockSpec((EMB, _BR), lambda c: (0, c))],
        out_specs=pl.BlockSpec((_BR // 4, 128), lambda c: (c, 0)),
        out_shape=jax.ShapeDtypeStruct((NROWS // 4, 128), jnp.float32),
    )(table.T)


_mesh = plsc.VectorSubcoreMesh(core_axis_name="c", subcore_axis_name="s")


@functools.partial(
    pl.kernel,
    out_type=jax.ShapeDtypeStruct((BATCH // CHUNK, CHUNK), jnp.float32),
    mesh=_mesh,
    compiler_params=pltpu.CompilerParams(
        needs_layout_passes=False, use_tc_tiling_on_sc=False),
    scratch_types=[
        pltpu.VMEM((NCHUNK, CHUNK), jnp.int32),
        pltpu.VMEM((NCHUNK, CHUNK), jnp.int32),
        pltpu.VMEM((NCHUNK, CHUNK), jnp.int32),
        pltpu.VMEM((NCHUNK, CHUNK), jnp.int32),
        pltpu.VMEM((2, CHUNK, 128), jnp.float32),
        pltpu.VMEM((2, CHUNK, 128), jnp.float32),
        pltpu.VMEM((NCHUNK, CHUNK), jnp.float32),
        pltpu.VMEM((NCHUNK, CHUNK), jnp.float32),
        pltpu.VMEM((NCHUNK, CHUNK), jnp.float32),
        pltpu.SemaphoreType.DMA,
        pltpu.SemaphoreType.DMA,
    ],
)
def _mf_sc_kernel(edge_ref, upk_ref, ipk_ref, ub_ref, ib_ref, out_ref,
                  uidx, iidx, usup, isup, urows, irows, ubv, ibv, outv,
                  sem_rows, sem_bias):
    wid = lax.axis_index("s") * 2 + lax.axis_index("c")
    row0 = wid * NCHUNK

    pltpu.sync_copy(edge_ref.at[0, pl.ds(row0, NCHUNK), :], uidx)
    pltpu.sync_copy(edge_ref.at[1, pl.ds(row0, NCHUNK), :], iidx)

    lane_iota = lax.iota(jnp.int32, LANES)
    for j in range(NCHUNK):
        for g in range(CHUNK // LANES):
            sl = pl.ds(g * LANES, LANES)
            usup[j, sl] = lax.shift_right_logical(uidx[j, sl], 2)
            isup[j, sl] = lax.shift_right_logical(iidx[j, sl], 2)

    bias_copies = []
    for j in range(NCHUNK):
        bias_copies.append(
            pltpu.async_copy(ub_ref.at[uidx.at[j]], ubv.at[j], sem_bias))
        bias_copies.append(
            pltpu.async_copy(ib_ref.at[iidx.at[j]], ibv.at[j], sem_bias))

    def fire(j):
        b = j % 2
        return (
            pltpu.async_copy(upk_ref.at[usup.at[j]], urows.at[b], sem_rows),
            pltpu.async_copy(ipk_ref.at[isup.at[j]], irows.at[b], sem_rows),
        )

    inflight = fire(0)
    for c in bias_copies:
        c.wait()

    for j in range(NCHUNK):
        cu, ci = inflight
        cu.wait()
        ci.wait()
        if j + 1 < NCHUNK:
            inflight = fire(j + 1)
        b = j % 2
        bsplat = jnp.full((LANES,), b, jnp.int32)

        def group(g, carry, j=j, bsplat=bsplat):
            sl = pl.ds(g * LANES, LANES)
            e_ids = g * LANES + lane_iota
            off_u = lax.shift_left(jnp.bitwise_and(uidx[j, sl], 3), 5)
            off_i = lax.shift_left(jnp.bitwise_and(iidx[j, sl], 3), 5)
            acc = ubv[j, sl] + ibv[j, sl]
            for d in range(EMB):
                uv = plsc.load_gather(urows, [bsplat, e_ids, off_u + d])
                iv = plsc.load_gather(irows, [bsplat, e_ids, off_i + d])
                acc = acc + uv * iv
            outv[j, sl] = acc
            return carry

        lax.fori_loop(0, CHUNK // LANES, group, 0)

    pltpu.sync_copy(outv, out_ref.at[pl.ds(row0, NCHUNK), :])


def kernel(edge_index, user_emb, item_emb, user_bias, item_bias):
    edge3 = edge_index.reshape(2, BATCH // CHUNK, CHUNK).astype(jnp.int32)
    upk = _pack_table(user_emb)
    ipk = _pack_table(item_emb)
    out = _mf_sc_kernel(edge3, upk, ipk,
                        user_bias.reshape(-1), item_bias.reshape(-1))
    return out.reshape(BATCH)


# speedup vs baseline: 1.1991x; 1.1991x over previous
import functools

import jax
import jax.numpy as jnp
from jax import lax
from jax.experimental import pallas as pl
from jax.experimental.pallas import tpu as pltpu
from jax.experimental.pallas import tpu_sc as plsc

EMB = 32
NROWS = 1000000
BATCH = 16384
NW = 32
B_PER_W = BATCH // NW
NCHUNK = 4
CHUNK = B_PER_W // NCHUNK
LANES = 16

_BR = 2048
_TC_GRID = (NROWS + _BR - 1) // _BR
_PK_ROWS = _TC_GRID * (_BR // 4)


def _detile_body(x_ref, o_ref):
    acc = None
    for q in range(4):
        xq = x_ref[:, q * (_BR // 4):(q + 1) * (_BR // 4)]
        eq = jnp.eye(EMB, 128, k=q * EMB, dtype=jnp.float32)
        p = lax.dot_general(xq, eq, (((0,), (0,)), ((), ())),
                            preferred_element_type=jnp.float32)
        acc = p if acc is None else acc + p
    o_ref[...] = acc


def _pack_table(table):
    return pl.pallas_call(
        _detile_body,
        grid=(_TC_GRID,),
        in_specs=[pl.BlockSpec((EMB, _BR), lambda c: (0, c))],
        out_specs=pl.BlockSpec((_BR // 4, 128), lambda c: (c, 0)),
        out_shape=jax.ShapeDtypeStruct((_PK_ROWS, 128), jnp.float32),
        compiler_params=pltpu.CompilerParams(fuse_transposed_lhs_in_matmul=True),
    )(table.T)


_mesh = plsc.VectorSubcoreMesh(core_axis_name="c", subcore_axis_name="s")


@functools.partial(
    pl.kernel,
    out_type=jax.ShapeDtypeStruct((BATCH // CHUNK, CHUNK), jnp.float32),
    mesh=_mesh,
    compiler_params=pltpu.CompilerParams(
        needs_layout_passes=False, use_tc_tiling_on_sc=False),
    scratch_types=[
        pltpu.VMEM((NCHUNK, CHUNK), jnp.int32),
        pltpu.VMEM((NCHUNK, CHUNK), jnp.int32),
        pltpu.VMEM((NCHUNK, CHUNK), jnp.int32),
        pltpu.VMEM((NCHUNK, CHUNK), jnp.int32),
        pltpu.VMEM((2, CHUNK, 128), jnp.float32),
        pltpu.VMEM((2, CHUNK, 128), jnp.float32),
        pltpu.VMEM((NCHUNK, CHUNK), jnp.float32),
        pltpu.VMEM((NCHUNK, CHUNK), jnp.float32),
        pltpu.VMEM((NCHUNK, CHUNK), jnp.float32),
        pltpu.SemaphoreType.DMA,
        pltpu.SemaphoreType.DMA,
    ],
)
def _mf_sc_kernel(edge_ref, upk_ref, ipk_ref, ub_ref, ib_ref, out_ref,
                  uidx, iidx, usup, isup, urows, irows, ubv, ibv, outv,
                  sem_rows, sem_bias):
    wid = lax.axis_index("s") * 2 + lax.axis_index("c")
    row0 = wid * NCHUNK

    pltpu.sync_copy(edge_ref.at[0, pl.ds(row0, NCHUNK), :], uidx)
    pltpu.sync_copy(edge_ref.at[1, pl.ds(row0, NCHUNK), :], iidx)

    lane_iota = lax.iota(jnp.int32, LANES)
    for j in range(NCHUNK):
        for g in range(CHUNK // LANES):
            sl = pl.ds(g * LANES, LANES)
            u = uidx[j, sl]
            i = iidx[j, sl]
            usup[j, sl] = jnp.bitwise_or(
                lax.shift_left(lax.shift_right_logical(u, 11), 9),
                jnp.bitwise_and(u, 511))
            isup[j, sl] = jnp.bitwise_or(
                lax.shift_left(lax.shift_right_logical(i, 11), 9),
                jnp.bitwise_and(i, 511))

    bias_copies = []
    for j in range(NCHUNK):
        bias_copies.append(
            pltpu.async_copy(ub_ref.at[uidx.at[j]], ubv.at[j], sem_bias))
        bias_copies.append(
            pltpu.async_copy(ib_ref.at[iidx.at[j]], ibv.at[j], sem_bias))

    def fire(j):
        b = j % 2
        return (
            pltpu.async_copy(upk_ref.at[usup.at[j]], urows.at[b], sem_rows),
            pltpu.async_copy(ipk_ref.at[isup.at[j]], irows.at[b], sem_rows),
        )

    inflight = fire(0)
    for c in bias_copies:
        c.wait()

    for j in range(NCHUNK):
        cu, ci = inflight
        cu.wait()
        ci.wait()
        if j + 1 < NCHUNK:
            inflight = fire(j + 1)
        b = j % 2
        bsplat = jnp.full((LANES,), b, jnp.int32)

        def group(g, carry, j=j, bsplat=bsplat):
            sl = pl.ds(g * LANES, LANES)
            e_ids = g * LANES + lane_iota
            off_u = lax.shift_left(
                jnp.bitwise_and(lax.shift_right_logical(uidx[j, sl], 9), 3), 5)
            off_i = lax.shift_left(
                jnp.bitwise_and(lax.shift_right_logical(iidx[j, sl], 9), 3), 5)
            acc = ubv[j, sl] + ibv[j, sl]
            for d in range(EMB):
                uv = plsc.load_gather(urows, [bsplat, e_ids, off_u + d])
                iv = plsc.load_gather(irows, [bsplat, e_ids, off_i + d])
                acc = acc + uv * iv
            outv[j, sl] = acc
            return carry

        lax.fori_loop(0, CHUNK // LANES, group, 0)

    pltpu.sync_copy(outv, out_ref.at[pl.ds(row0, NCHUNK), :])


def kernel(edge_index, user_emb, item_emb, user_bias, item_bias):
    edge3 = edge_index.reshape(2, BATCH // CHUNK, CHUNK).astype(jnp.int32)
    upk = _pack_table(user_emb)
    ipk = _pack_table(item_emb)
    out = _mf_sc_kernel(edge3, upk, ipk,
                        user_bias.reshape(-1), item_bias.reshape(-1))
    return out.reshape(BATCH)


# speedup vs baseline: 2.4343x; 2.0300x over previous
import functools

import jax
import jax.numpy as jnp
from jax import lax
from jax.experimental import pallas as pl
from jax.experimental.pallas import tpu as pltpu
from jax.experimental.pallas import tpu_sc as plsc

EMB = 32
NROWS = 1000000
BATCH = 16384
NW = 32
B_PER_W = BATCH // NW
NCHUNK = 4
CHUNK = B_PER_W // NCHUNK
LANES = 16

_BR = 8192
_TC_GRID = (NROWS + _BR - 1) // _BR
_PK_ROWS = _TC_GRID * (_BR // 4)
_QR = _BR // 4


def _detile_body(x_ref, o_ref):
    acc = None
    for q in range(4):
        xq = x_ref[:, q * _QR:(q + 1) * _QR].astype(jnp.bfloat16)
        eq = jnp.eye(EMB, 128, k=q * EMB, dtype=jnp.bfloat16)
        p = lax.dot_general(xq, eq, (((0,), (0,)), ((), ())),
                            preferred_element_type=jnp.float32)
        acc = p if acc is None else acc + p
    o_ref[...] = acc


def _pack_table(table):
    return pl.pallas_call(
        _detile_body,
        grid=(_TC_GRID,),
        in_specs=[pl.BlockSpec((EMB, _BR), lambda c: (0, c))],
        out_specs=pl.BlockSpec((_BR // 4, 128), lambda c: (c, 0)),
        out_shape=jax.ShapeDtypeStruct((_PK_ROWS, 128), jnp.float32),
        compiler_params=pltpu.CompilerParams(fuse_transposed_lhs_in_matmul=True),
    )(table.T)


_mesh = plsc.VectorSubcoreMesh(core_axis_name="c", subcore_axis_name="s")


@functools.partial(
    pl.kernel,
    out_type=jax.ShapeDtypeStruct((BATCH // CHUNK, CHUNK), jnp.float32),
    mesh=_mesh,
    compiler_params=pltpu.CompilerParams(
        needs_layout_passes=False, use_tc_tiling_on_sc=False),
    scratch_types=[
        pltpu.VMEM((NCHUNK, CHUNK), jnp.int32),
        pltpu.VMEM((NCHUNK, CHUNK), jnp.int32),
        pltpu.VMEM((NCHUNK, CHUNK), jnp.int32),
        pltpu.VMEM((NCHUNK, CHUNK), jnp.int32),
        pltpu.VMEM((2, CHUNK, 128), jnp.float32),
        pltpu.VMEM((2, CHUNK, 128), jnp.float32),
        pltpu.VMEM((NCHUNK, CHUNK), jnp.float32),
        pltpu.VMEM((NCHUNK, CHUNK), jnp.float32),
        pltpu.VMEM((NCHUNK, CHUNK), jnp.float32),
        pltpu.SemaphoreType.DMA,
        pltpu.SemaphoreType.DMA,
    ],
)
def _mf_sc_kernel(edge_ref, upk_ref, ipk_ref, ub_ref, ib_ref, out_ref,
                  uidx, iidx, usup, isup, urows, irows, ubv, ibv, outv,
                  sem_rows, sem_bias):
    wid = lax.axis_index("s") * 2 + lax.axis_index("c")
    row0 = wid * NCHUNK

    pltpu.sync_copy(edge_ref.at[0, pl.ds(row0, NCHUNK), :], uidx)
    pltpu.sync_copy(edge_ref.at[1, pl.ds(row0, NCHUNK), :], iidx)

    lane_iota = lax.iota(jnp.int32, LANES)
    for j in range(NCHUNK):
        for g in range(CHUNK // LANES):
            sl = pl.ds(g * LANES, LANES)
            u = uidx[j, sl]
            i = iidx[j, sl]
            usup[j, sl] = jnp.bitwise_or(
                lax.shift_left(lax.shift_right_logical(u, 13), 11),
                jnp.bitwise_and(u, 2047))
            isup[j, sl] = jnp.bitwise_or(
                lax.shift_left(lax.shift_right_logical(i, 13), 11),
                jnp.bitwise_and(i, 2047))

    bias_copies = []
    for j in range(NCHUNK):
        bias_copies.append(
            pltpu.async_copy(ub_ref.at[uidx.at[j]], ubv.at[j], sem_bias))
        bias_copies.append(
            pltpu.async_copy(ib_ref.at[iidx.at[j]], ibv.at[j], sem_bias))

    def fire(j):
        b = j % 2
        return (
            pltpu.async_copy(upk_ref.at[usup.at[j]], urows.at[b], sem_rows),
            pltpu.async_copy(ipk_ref.at[isup.at[j]], irows.at[b], sem_rows),
        )

    inflight = fire(0)
    for c in bias_copies:
        c.wait()

    for j in range(NCHUNK):
        cu, ci = inflight
        cu.wait()
        ci.wait()
        if j + 1 < NCHUNK:
            inflight = fire(j + 1)
        b = j % 2
        bsplat = jnp.full((LANES,), b, jnp.int32)

        def group(g, carry, j=j, bsplat=bsplat):
            sl = pl.ds(g * LANES, LANES)
            e_ids = g * LANES + lane_iota
            off_u = lax.shift_left(
                jnp.bitwise_and(lax.shift_right_logical(uidx[j, sl], 11), 3), 5)
            off_i = lax.shift_left(
                jnp.bitwise_and(lax.shift_right_logical(iidx[j, sl], 11), 3), 5)
            acc = ubv[j, sl] + ibv[j, sl]
            for d in range(EMB):
                uv = plsc.load_gather(urows, [bsplat, e_ids, off_u + d])
                iv = plsc.load_gather(irows, [bsplat, e_ids, off_i + d])
                acc = acc + uv * iv
            outv[j, sl] = acc
            return carry

        lax.fori_loop(0, CHUNK // LANES, group, 0)

    pltpu.sync_copy(outv, out_ref.at[pl.ds(row0, NCHUNK), :])


def kernel(edge_index, user_emb, item_emb, user_bias, item_bias):
    edge3 = edge_index.reshape(2, BATCH // CHUNK, CHUNK).astype(jnp.int32)
    upk = _pack_table(user_emb)
    ipk = _pack_table(item_emb)
    out = _mf_sc_kernel(edge3, upk, ipk,
                        user_bias.reshape(-1), item_bias.reshape(-1))
    return out.reshape(BATCH)


# speedup vs baseline: 2.9454x; 1.2100x over previous
import functools

import jax
import jax.numpy as jnp
from jax import lax
from jax.experimental import pallas as pl
from jax.experimental.pallas import tpu as pltpu
from jax.experimental.pallas import tpu_sc as plsc

EMB = 32
NROWS = 1000000
BATCH = 16384
NW = 32
B_PER_W = BATCH // NW
NCHUNK = 4
CHUNK = B_PER_W // NCHUNK
LANES = 16

_BR = 8192
_TC_GRID = (NROWS + _BR - 1) // _BR
_PK_ROWS = _TC_GRID * (_BR // 4)
_QR = _BR // 4


def _pack_one(x):
    acc = None
    for q in range(4):
        xq = x[:, q * _QR:(q + 1) * _QR].astype(jnp.bfloat16)
        eq = jnp.eye(EMB, 128, k=q * EMB, dtype=jnp.bfloat16)
        p = lax.dot_general(xq, eq, (((0,), (0,)), ((), ())),
                            preferred_element_type=jnp.float32)
        acc = p if acc is None else acc + p
    return acc


def _detile_body(xu_ref, xi_ref, ou_ref, oi_ref):
    ou_ref[...] = _pack_one(xu_ref[...])
    oi_ref[...] = _pack_one(xi_ref[...])


def _pack_tables(user_emb, item_emb):
    spec_in = pl.BlockSpec((EMB, _BR), lambda c: (0, c))
    spec_out = pl.BlockSpec((_BR // 4, 128), lambda c: (c, 0))
    oshape = jax.ShapeDtypeStruct((_PK_ROWS, 128), jnp.float32)
    return pl.pallas_call(
        _detile_body,
        grid=(_TC_GRID,),
        in_specs=[spec_in, spec_in],
        out_specs=[spec_out, spec_out],
        out_shape=[oshape, oshape],
        compiler_params=pltpu.CompilerParams(fuse_transposed_lhs_in_matmul=True),
    )(user_emb.T, item_emb.T)


_mesh = plsc.VectorSubcoreMesh(core_axis_name="c", subcore_axis_name="s")


@functools.partial(
    pl.kernel,
    out_type=jax.ShapeDtypeStruct((BATCH // CHUNK, CHUNK), jnp.float32),
    mesh=_mesh,
    compiler_params=pltpu.CompilerParams(
        needs_layout_passes=False, use_tc_tiling_on_sc=False),
    scratch_types=[
        pltpu.VMEM((NCHUNK, CHUNK), jnp.int32),
        pltpu.VMEM((NCHUNK, CHUNK), jnp.int32),
        pltpu.VMEM((NCHUNK, CHUNK), jnp.int32),
        pltpu.VMEM((NCHUNK, CHUNK), jnp.int32),
        pltpu.VMEM((2, CHUNK, 128), jnp.float32),
        pltpu.VMEM((2, CHUNK, 128), jnp.float32),
        pltpu.VMEM((NCHUNK, CHUNK), jnp.float32),
        pltpu.VMEM((NCHUNK, CHUNK), jnp.float32),
        pltpu.VMEM((NCHUNK, CHUNK), jnp.float32),
        pltpu.SemaphoreType.DMA,
        pltpu.SemaphoreType.DMA,
    ],
)
def _mf_sc_kernel(edge_ref, upk_ref, ipk_ref, ub_ref, ib_ref, out_ref,
                  uidx, iidx, usup, isup, urows, irows, ubv, ibv, outv,
                  sem_rows, sem_bias):
    wid = lax.axis_index("s") * 2 + lax.axis_index("c")
    row0 = wid * NCHUNK

    pltpu.sync_copy(edge_ref.at[0, pl.ds(row0, NCHUNK), :], uidx)
    pltpu.sync_copy(edge_ref.at[1, pl.ds(row0, NCHUNK), :], iidx)

    lane_iota = lax.iota(jnp.int32, LANES)
    for j in range(NCHUNK):
        for g in range(CHUNK // LANES):
            sl = pl.ds(g * LANES, LANES)
            u = uidx[j, sl]
            i = iidx[j, sl]
            usup[j, sl] = jnp.bitwise_or(
                lax.shift_left(lax.shift_right_logical(u, 13), 11),
                jnp.bitwise_and(u, 2047))
            isup[j, sl] = jnp.bitwise_or(
                lax.shift_left(lax.shift_right_logical(i, 13), 11),
                jnp.bitwise_and(i, 2047))

    bias_copies = []
    for j in range(NCHUNK):
        bias_copies.append(
            pltpu.async_copy(ub_ref.at[uidx.at[j]], ubv.at[j], sem_bias))
        bias_copies.append(
            pltpu.async_copy(ib_ref.at[iidx.at[j]], ibv.at[j], sem_bias))

    def fire(j):
        b = j % 2
        return (
            pltpu.async_copy(upk_ref.at[usup.at[j]], urows.at[b], sem_rows),
            pltpu.async_copy(ipk_ref.at[isup.at[j]], irows.at[b], sem_rows),
        )

    inflight = fire(0)
    for c in bias_copies:
        c.wait()

    for j in range(NCHUNK):
        cu, ci = inflight
        cu.wait()
        ci.wait()
        if j + 1 < NCHUNK:
            inflight = fire(j + 1)
        b = j % 2
        bsplat = jnp.full((LANES,), b, jnp.int32)

        def group(g, carry, j=j, bsplat=bsplat):
            sl = pl.ds(g * LANES, LANES)
            e_ids = g * LANES + lane_iota
            off_u = lax.shift_left(
                jnp.bitwise_and(lax.shift_right_logical(uidx[j, sl], 11), 3), 5)
            off_i = lax.shift_left(
                jnp.bitwise_and(lax.shift_right_logical(iidx[j, sl], 11), 3), 5)
            acc = ubv[j, sl] + ibv[j, sl]
            for d in range(EMB):
                uv = plsc.load_gather(urows, [bsplat, e_ids, off_u + d])
                iv = plsc.load_gather(irows, [bsplat, e_ids, off_i + d])
                acc = acc + uv * iv
            outv[j, sl] = acc
            return carry

        lax.fori_loop(0, CHUNK // LANES, group, 0)

    pltpu.sync_copy(outv, out_ref.at[pl.ds(row0, NCHUNK), :])


def kernel(edge_index, user_emb, item_emb, user_bias, item_bias):
    edge3 = edge_index.reshape(2, BATCH // CHUNK, CHUNK).astype(jnp.int32)
    upk, ipk = _pack_tables(user_emb, item_emb)
    out = _mf_sc_kernel(edge3, upk, ipk,
                        user_bias.reshape(-1), item_bias.reshape(-1))
    return out.reshape(BATCH)


# speedup vs baseline: 3.3062x; 1.1225x over previous
import functools

import jax
import jax.numpy as jnp
from jax import lax
from jax.experimental import pallas as pl
from jax.experimental.pallas import tpu as pltpu
from jax.experimental.pallas import tpu_sc as plsc

EMB = 32
NROWS = 1000000
BATCH = 16384
NW = 32
B_PER_W = BATCH // NW
NCHUNK = 4
CHUNK = B_PER_W // NCHUNK
LANES = 16

_BR = 16384
_TC_GRID = (NROWS + _BR - 1) // _BR
_PK_ROWS = _TC_GRID * (_BR // 4)
_QR = _BR // 4


def _pack_one(x):
    acc = None
    for q in range(4):
        xq = x[:, q * _QR:(q + 1) * _QR].astype(jnp.bfloat16)
        eq = jnp.eye(EMB, 128, k=q * EMB, dtype=jnp.bfloat16)
        p = lax.dot_general(xq, eq, (((0,), (0,)), ((), ())),
                            preferred_element_type=jnp.float32)
        acc = p if acc is None else acc + p
    return acc


def _detile_body(xu_ref, xi_ref, ou_ref, oi_ref):
    ou_ref[...] = _pack_one(xu_ref[...])
    oi_ref[...] = _pack_one(xi_ref[...])


def _pack_tables(user_emb, item_emb):
    spec_in = pl.BlockSpec((EMB, _BR), lambda c: (0, c))
    spec_out = pl.BlockSpec((_BR // 4, 128), lambda c: (c, 0))
    oshape = jax.ShapeDtypeStruct((_PK_ROWS, 128), jnp.float32)
    return pl.pallas_call(
        _detile_body,
        grid=(_TC_GRID,),
        in_specs=[spec_in, spec_in],
        out_specs=[spec_out, spec_out],
        out_shape=[oshape, oshape],
        compiler_params=pltpu.CompilerParams(fuse_transposed_lhs_in_matmul=True),
    )(user_emb.T, item_emb.T)


_mesh = plsc.VectorSubcoreMesh(core_axis_name="c", subcore_axis_name="s")


@functools.partial(
    pl.kernel,
    out_type=jax.ShapeDtypeStruct((BATCH // CHUNK, CHUNK), jnp.float32),
    mesh=_mesh,
    compiler_params=pltpu.CompilerParams(
        needs_layout_passes=False, use_tc_tiling_on_sc=False),
    scratch_types=[
        pltpu.VMEM((NCHUNK, CHUNK), jnp.int32),
        pltpu.VMEM((NCHUNK, CHUNK), jnp.int32),
        pltpu.VMEM((NCHUNK, CHUNK), jnp.int32),
        pltpu.VMEM((NCHUNK, CHUNK), jnp.int32),
        pltpu.VMEM((2, CHUNK, 128), jnp.float32),
        pltpu.VMEM((2, CHUNK, 128), jnp.float32),
        pltpu.VMEM((NCHUNK, CHUNK), jnp.float32),
        pltpu.VMEM((NCHUNK, CHUNK), jnp.float32),
        pltpu.VMEM((NCHUNK, CHUNK), jnp.float32),
        pltpu.SemaphoreType.DMA,
        pltpu.SemaphoreType.DMA,
    ],
)
def _mf_sc_kernel(edge_ref, upk_ref, ipk_ref, ub_ref, ib_ref, out_ref,
                  uidx, iidx, usup, isup, urows, irows, ubv, ibv, outv,
                  sem_rows, sem_bias):
    wid = lax.axis_index("s") * 2 + lax.axis_index("c")
    row0 = wid * NCHUNK

    pltpu.sync_copy(edge_ref.at[0, pl.ds(row0, NCHUNK), :], uidx)
    pltpu.sync_copy(edge_ref.at[1, pl.ds(row0, NCHUNK), :], iidx)

    lane_iota = lax.iota(jnp.int32, LANES)
    for j in range(NCHUNK):
        for g in range(CHUNK // LANES):
            sl = pl.ds(g * LANES, LANES)
            u = uidx[j, sl]
            i = iidx[j, sl]
            usup[j, sl] = jnp.bitwise_or(
                lax.shift_left(lax.shift_right_logical(u, 14), 12),
                jnp.bitwise_and(u, 4095))
            isup[j, sl] = jnp.bitwise_or(
                lax.shift_left(lax.shift_right_logical(i, 14), 12),
                jnp.bitwise_and(i, 4095))

    bias_copies = []
    for j in range(NCHUNK):
        bias_copies.append(
            pltpu.async_copy(ub_ref.at[uidx.at[j]], ubv.at[j], sem_bias))
        bias_copies.append(
            pltpu.async_copy(ib_ref.at[iidx.at[j]], ibv.at[j], sem_bias))

    def fire(j):
        b = j % 2
        return (
            pltpu.async_copy(upk_ref.at[usup.at[j]], urows.at[b], sem_rows),
            pltpu.async_copy(ipk_ref.at[isup.at[j]], irows.at[b], sem_rows),
        )

    inflight = fire(0)
    for c in bias_copies:
        c.wait()

    for j in range(NCHUNK):
        cu, ci = inflight
        cu.wait()
        ci.wait()
        if j + 1 < NCHUNK:
            inflight = fire(j + 1)
        b = j % 2
        bsplat = jnp.full((LANES,), b, jnp.int32)

        def group(g, carry, j=j, bsplat=bsplat):
            sl = pl.ds(g * LANES, LANES)
            e_ids = g * LANES + lane_iota
            off_u = lax.shift_left(
                jnp.bitwise_and(lax.shift_right_logical(uidx[j, sl], 12), 3), 5)
            off_i = lax.shift_left(
                jnp.bitwise_and(lax.shift_right_logical(iidx[j, sl], 12), 3), 5)
            acc = ubv[j, sl] + ibv[j, sl]
            for d in range(EMB):
                uv = plsc.load_gather(urows, [bsplat, e_ids, off_u + d])
                iv = plsc.load_gather(irows, [bsplat, e_ids, off_i + d])
                acc = acc + uv * iv
            outv[j, sl] = acc
            return carry

        lax.fori_loop(0, CHUNK // LANES, group, 0)

    pltpu.sync_copy(outv, out_ref.at[pl.ds(row0, NCHUNK), :])


def kernel(edge_index, user_emb, item_emb, user_bias, item_bias):
    edge3 = edge_index.reshape(2, BATCH // CHUNK, CHUNK).astype(jnp.int32)
    upk, ipk = _pack_tables(user_emb, item_emb)
    out = _mf_sc_kernel(edge3, upk, ipk,
                        user_bias.reshape(-1), item_bias.reshape(-1))
    return out.reshape(BATCH)


# speedup vs baseline: 3.5176x; 1.0639x over previous
import functools

import jax
import jax.numpy as jnp
from jax import lax
from jax.experimental import pallas as pl
from jax.experimental.pallas import tpu as pltpu
from jax.experimental.pallas import tpu_sc as plsc

EMB = 32
NROWS = 1000000
BATCH = 16384
NW = 32
B_PER_W = BATCH // NW
NCHUNK = 4
CHUNK = B_PER_W // NCHUNK
LANES = 16

_BR = 32768
_TC_GRID = (NROWS + _BR - 1) // _BR
_PK_ROWS = _TC_GRID * (_BR // 4)
_QR = _BR // 4


def _pack_one(x):
    acc = None
    for q in range(4):
        xq = x[:, q * _QR:(q + 1) * _QR].astype(jnp.bfloat16)
        eq = jnp.eye(EMB, 128, k=q * EMB, dtype=jnp.bfloat16)
        p = lax.dot_general(xq, eq, (((0,), (0,)), ((), ())),
                            preferred_element_type=jnp.float32)
        acc = p if acc is None else acc + p
    return acc


def _detile_body(xu_ref, xi_ref, ou_ref, oi_ref):
    ou_ref[...] = _pack_one(xu_ref[...])
    oi_ref[...] = _pack_one(xi_ref[...])


def _pack_tables(user_emb, item_emb):
    spec_in = pl.BlockSpec((EMB, _BR), lambda c: (0, c))
    spec_out = pl.BlockSpec((_BR // 4, 128), lambda c: (c, 0))
    oshape = jax.ShapeDtypeStruct((_PK_ROWS, 128), jnp.float32)
    return pl.pallas_call(
        _detile_body,
        grid=(_TC_GRID,),
        in_specs=[spec_in, spec_in],
        out_specs=[spec_out, spec_out],
        out_shape=[oshape, oshape],
        compiler_params=pltpu.CompilerParams(fuse_transposed_lhs_in_matmul=True),
    )(user_emb.T, item_emb.T)


_mesh = plsc.VectorSubcoreMesh(core_axis_name="c", subcore_axis_name="s")


@functools.partial(
    pl.kernel,
    out_type=jax.ShapeDtypeStruct((BATCH // CHUNK, CHUNK), jnp.float32),
    mesh=_mesh,
    compiler_params=pltpu.CompilerParams(
        needs_layout_passes=False, use_tc_tiling_on_sc=False),
    scratch_types=[
        pltpu.VMEM((NCHUNK, CHUNK), jnp.int32),
        pltpu.VMEM((NCHUNK, CHUNK), jnp.int32),
        pltpu.VMEM((NCHUNK, CHUNK), jnp.int32),
        pltpu.VMEM((NCHUNK, CHUNK), jnp.int32),
        pltpu.VMEM((2, CHUNK, 128), jnp.float32),
        pltpu.VMEM((2, CHUNK, 128), jnp.float32),
        pltpu.VMEM((NCHUNK, CHUNK), jnp.float32),
        pltpu.VMEM((NCHUNK, CHUNK), jnp.float32),
        pltpu.VMEM((NCHUNK, CHUNK), jnp.float32),
        pltpu.SemaphoreType.DMA,
        pltpu.SemaphoreType.DMA,
    ],
)
def _mf_sc_kernel(edge_ref, upk_ref, ipk_ref, ub_ref, ib_ref, out_ref,
                  uidx, iidx, usup, isup, urows, irows, ubv, ibv, outv,
                  sem_rows, sem_bias):
    wid = lax.axis_index("s") * 2 + lax.axis_index("c")
    row0 = wid * NCHUNK

    pltpu.sync_copy(edge_ref.at[0, pl.ds(row0, NCHUNK), :], uidx)
    pltpu.sync_copy(edge_ref.at[1, pl.ds(row0, NCHUNK), :], iidx)

    lane_iota = lax.iota(jnp.int32, LANES)
    for j in range(NCHUNK):
        for g in range(CHUNK // LANES):
            sl = pl.ds(g * LANES, LANES)
            u = uidx[j, sl]
            i = iidx[j, sl]
            usup[j, sl] = jnp.bitwise_or(
                lax.shift_left(lax.shift_right_logical(u, 15), 13),
                jnp.bitwise_and(u, 8191))
            isup[j, sl] = jnp.bitwise_or(
                lax.shift_left(lax.shift_right_logical(i, 15), 13),
                jnp.bitwise_and(i, 8191))

    bias_copies = []
    for j in range(NCHUNK):
        bias_copies.append(
            pltpu.async_copy(ub_ref.at[uidx.at[j]], ubv.at[j], sem_bias))
        bias_copies.append(
            pltpu.async_copy(ib_ref.at[iidx.at[j]], ibv.at[j], sem_bias))

    def fire(j):
        b = j % 2
        return (
            pltpu.async_copy(upk_ref.at[usup.at[j]], urows.at[b], sem_rows),
            pltpu.async_copy(ipk_ref.at[isup.at[j]], irows.at[b], sem_rows),
        )

    inflight = fire(0)
    for c in bias_copies:
        c.wait()

    for j in range(NCHUNK):
        cu, ci = inflight
        cu.wait()
        ci.wait()
        if j + 1 < NCHUNK:
            inflight = fire(j + 1)
        b = j % 2
        bsplat = jnp.full((LANES,), b, jnp.int32)

        def group(g, carry, j=j, bsplat=bsplat):
            sl = pl.ds(g * LANES, LANES)
            e_ids = g * LANES + lane_iota
            off_u = lax.shift_left(
                jnp.bitwise_and(lax.shift_right_logical(uidx[j, sl], 13), 3), 5)
            off_i = lax.shift_left(
                jnp.bitwise_and(lax.shift_right_logical(iidx[j, sl], 13), 3), 5)
            acc = ubv[j, sl] + ibv[j, sl]
            for d in range(EMB):
                uv = plsc.load_gather(urows, [bsplat, e_ids, off_u + d])
                iv = plsc.load_gather(irows, [bsplat, e_ids, off_i + d])
                acc = acc + uv * iv
            outv[j, sl] = acc
            return carry

        lax.fori_loop(0, CHUNK // LANES, group, 0)

    pltpu.sync_copy(outv, out_ref.at[pl.ds(row0, NCHUNK), :])


def kernel(edge_index, user_emb, item_emb, user_bias, item_bias):
    edge3 = edge_index.reshape(2, BATCH // CHUNK, CHUNK).astype(jnp.int32)
    upk, ipk = _pack_tables(user_emb, item_emb)
    out = _mf_sc_kernel(edge3, upk, ipk,
                        user_bias.reshape(-1), item_bias.reshape(-1))
    return out.reshape(BATCH)
